# Initial kernel scaffold; baseline (speedup 1.0000x reference)
import jax
import jax.numpy as jnp
from jax.experimental import pallas as pl


def kernel(scores):
    raise NotImplementedError("write your pallas kernel here")


# speedup vs baseline: 1.5499x; 1.5499x over previous
import jax
import jax.numpy as jnp
from jax.experimental import pallas as pl

B = 8
N = 2048
BI = 256
NI = N // BI


def _soft_sort_body(s_ref, t_ref, o_ref):
    s = s_ref[:].reshape(1, N)
    t = t_ref[:].reshape(BI, 1)
    e = jnp.exp(-jnp.abs(s - t))
    recip = 1.0 / jnp.sum(e, axis=1, keepdims=True)
    o_ref[:] = (e * recip).reshape(1, BI, N)


def kernel(scores):
    sorted_s = jnp.sort(scores, axis=-1)
    srow = scores.reshape(B, 1, N)
    tcol = sorted_s.reshape(B, NI, BI, 1)
    return pl.pallas_call(
        _soft_sort_body,
        grid=(B, NI),
        in_specs=[
            pl.BlockSpec((1, 1, N), lambda b, i: (b, 0, 0)),
            pl.BlockSpec((1, 1, BI, 1), lambda b, i: (b, i, 0, 0)),
        ],
        out_specs=pl.BlockSpec((1, BI, N), lambda b, i: (b, i, 0)),
        out_shape=jax.ShapeDtypeStruct((B, N, N), jnp.float32),
    )(srow, tcol)


# speedup vs baseline: 2.2964x; 1.4817x over previous
import jax
import jax.numpy as jnp
from jax.experimental import pallas as pl

B = 8
N = 2048
BI = 2048
NI = N // BI


def _soft_sort_body(s_ref, t_ref, o_ref):
    s = s_ref[:].reshape(1, N)
    t = t_ref[:].reshape(BI, 1)
    e = jnp.exp(-jnp.abs(s - t))
    recip = 1.0 / jnp.sum(e, axis=1, keepdims=True)
    o_ref[:] = (e * recip).reshape(1, BI, N)


def kernel(scores):
    sorted_s = jnp.sort(scores, axis=-1)
    srow = scores.reshape(B, 1, N)
    tcol = sorted_s.reshape(B, NI, BI, 1)
    return pl.pallas_call(
        _soft_sort_body,
        grid=(B, NI),
        in_specs=[
            pl.BlockSpec((1, 1, N), lambda b, i: (b, 0, 0)),
            pl.BlockSpec((1, 1, BI, 1), lambda b, i: (b, i, 0, 0)),
        ],
        out_specs=pl.BlockSpec((1, BI, N), lambda b, i: (b, i, 0)),
        out_shape=jax.ShapeDtypeStruct((B, N, N), jnp.float32),
    )(srow, tcol)


# speedup vs baseline: 2.3127x; 1.0071x over previous
import jax
import jax.numpy as jnp
from jax.experimental import pallas as pl
from jax.experimental.pallas import tpu as pltpu

B = 8
N = 2048
BI = 2048
NI = N // BI


def _soft_sort_body(s_ref, t_ref, o_ref):
    s = s_ref[:].reshape(1, N)
    t = t_ref[:].reshape(BI, 1)
    e = jnp.exp(-jnp.abs(s - t))
    recip = 1.0 / jnp.sum(e, axis=1, keepdims=True)
    o_ref[:] = (e * recip).reshape(1, BI, N)


def kernel(scores):
    sorted_s = jnp.sort(scores, axis=-1)
    srow = scores.reshape(B, 1, N)
    tcol = sorted_s.reshape(B, NI, BI, 1)
    return pl.pallas_call(
        _soft_sort_body,
        grid=(B, NI),
        in_specs=[
            pl.BlockSpec((1, 1, N), lambda b, i: (b, 0, 0)),
            pl.BlockSpec((1, 1, BI, 1), lambda b, i: (b, i, 0, 0)),
        ],
        out_specs=pl.BlockSpec((1, BI, N), lambda b, i: (b, i, 0)),
        out_shape=jax.ShapeDtypeStruct((B, N, N), jnp.float32),
        compiler_params=pltpu.CompilerParams(
            dimension_semantics=("parallel", "parallel"),
        ),
    )(srow, tcol)
